# baseline (speedup 1.0000x reference)
import functools

import jax
import jax.numpy as jnp
from jax import lax
from jax.experimental import pallas as pl
from jax.experimental.pallas import tpu as pltpu
from jax.experimental.pallas import tpu_sc as plsc

CB = 8192
DIM = 256
FLAT = 16384
HALF = FLAT // 2
BM = 1024
NSTEPS_H = HALF // BM
NUMEL = FLAT * DIM

NCH = 8
CHN = CB // NCH


def _dist_argmin_body(x_ref, emb_ref, xsq_ref, esq_ref, locf_ref, idx_ref,
                      loss_ref):
    i = pl.program_id(0)

    @pl.when(i == 0)
    def _init():
        loss_ref[...] = jnp.zeros((1, 1), jnp.float32)

    x = x_ref[...]
    x2 = x + x
    xsq = xsq_ref[...]
    runmin = jnp.full((BM, 1), jnp.inf, jnp.float32)
    runidx = jnp.zeros((BM, 1), jnp.int32)
    for c in range(NCH):
        dot2 = lax.dot_general(
            x2, emb_ref[pl.ds(c * CHN, CHN), :],
            dimension_numbers=(((1,), (1,)), ((), ())),
            preferred_element_type=jnp.float32)
        dist = (xsq - dot2) + esq_ref[:, pl.ds(c * CHN, CHN)]
        cmin = jnp.min(dist, axis=1, keepdims=True)
        fidx = jnp.min(
            jnp.where(dist == cmin, locf_ref[:, pl.ds(c * CHN, CHN)],
                      jnp.inf),
            axis=1, keepdims=True)
        cidx = fidx.astype(jnp.int32)
        take = cmin < runmin
        runidx = jnp.where(take, cidx, runidx)
        runmin = jnp.where(take, cmin, runmin)
    idx_ref[...] = runidx
    loss_ref[...] = loss_ref[...] + jnp.sum(runmin, axis=0, keepdims=True)

    @pl.when(i == NSTEPS_H - 1)
    def _fin():
        loss_ref[...] = loss_ref[...] * (2.0 / NUMEL)


def _make_dist_argmin(part):
    off = part * NSTEPS_H
    return pl.pallas_call(
        _dist_argmin_body,
        grid=(NSTEPS_H,),
        in_specs=[
            pl.BlockSpec((BM, DIM), lambda i: (i + off, 0)),
            pl.BlockSpec((CB, DIM), lambda i: (0, 0)),
            pl.BlockSpec((BM, 1), lambda i: (i + off, 0)),
            pl.BlockSpec((1, CB), lambda i: (0, 0)),
            pl.BlockSpec((1, CB), lambda i: (0, 0)),
        ],
        out_specs=[
            pl.BlockSpec((BM, 1), lambda i: (i, 0)),
            pl.BlockSpec((1, 1), lambda i: (0, 0)),
        ],
        out_shape=[
            jax.ShapeDtypeStruct((HALF, 1), jnp.int32),
            jax.ShapeDtypeStruct((1, 1), jnp.float32),
        ],
    )


_dist_argmin_0 = _make_dist_argmin(0)
_dist_argmin_1 = _make_dist_argmin(1)


_NW = 32
_BPW = HALF // _NW
_CH = 128

_sc_mesh = plsc.VectorSubcoreMesh(core_axis_name="c", subcore_axis_name="s")


@functools.partial(
    pl.kernel,
    mesh=_sc_mesh,
    out_type=jax.ShapeDtypeStruct((HALF, DIM), jnp.float32),
    scratch_types=[
        pltpu.VMEM((_CH,), jnp.int32),
        pltpu.VMEM((_CH,), jnp.int32),
        pltpu.VMEM((_CH, DIM), jnp.float32),
        pltpu.VMEM((_CH, DIM), jnp.float32),
        pltpu.SemaphoreType.DMA,
        pltpu.SemaphoreType.DMA,
        pltpu.SemaphoreType.DMA,
        pltpu.SemaphoreType.DMA,
    ],
)
def _sc_gather(emb_hbm, idx_hbm, out_hbm, idxa, idxb, rowsa, rowsb,
               sa, sb, soa, sob):
    wid = lax.axis_index("s") * 2 + lax.axis_index("c")
    base = wid * _BPW
    c0, c1 = base, base + _CH
    pltpu.sync_copy(idx_hbm.at[pl.ds(c0, _CH)], idxa)
    g0 = pltpu.async_copy(emb_hbm.at[idxa], rowsa, sa)
    pltpu.sync_copy(idx_hbm.at[pl.ds(c1, _CH)], idxb)
    g1 = pltpu.async_copy(emb_hbm.at[idxb], rowsb, sb)
    g0.wait()
    o0 = pltpu.async_copy(rowsa, out_hbm.at[pl.ds(c0, _CH)], soa)
    g1.wait()
    o1 = pltpu.async_copy(rowsb, out_hbm.at[pl.ds(c1, _CH)], sob)
    o0.wait()
    o1.wait()


def kernel(x, embedding):
    flat = x.reshape(FLAT, DIM)
    xsq = jnp.sum(flat * flat, axis=1, keepdims=True)
    esq = jnp.sum(embedding * embedding, axis=1).reshape(1, CB)
    locf = jnp.arange(CB, dtype=jnp.float32).reshape(1, CB)
    idx0, loss0 = _dist_argmin_0(flat, embedding, xsq, esq, locf)
    q0 = _sc_gather(embedding, idx0.reshape(HALF))
    idx1, loss1 = _dist_argmin_1(flat, embedding, xsq, esq, locf)
    q1 = _sc_gather(embedding, idx1.reshape(HALF))
    quantized_st = jnp.concatenate([q0, q1], axis=0).reshape(x.shape)
    indices = jnp.concatenate([idx0, idx1], axis=0).reshape(x.shape[:-1])
    loss = (loss0 + loss1).reshape(())
    return quantized_st, indices, loss


# speedup vs baseline: 1.0659x; 1.0659x over previous
import functools

import jax
import jax.numpy as jnp
from jax import lax
from jax.experimental import pallas as pl
from jax.experimental.pallas import tpu as pltpu
from jax.experimental.pallas import tpu_sc as plsc

CB = 8192
DIM = 256
FLAT = 16384
BM = 1024
NSTEPS = FLAT // BM
NUMEL = FLAT * DIM

NCH = 8
CHN = CB // NCH


def _dist_argmin_body(x_ref, emb_ref, xsq_ref, esq_ref, locf_ref, idx_ref,
                      loss_ref):
    i = pl.program_id(0)

    @pl.when(i == 0)
    def _init():
        loss_ref[...] = jnp.zeros((1, 1), jnp.float32)

    x = x_ref[...]
    x2 = x + x
    xsq = xsq_ref[...]
    runmin = jnp.full((BM, 1), jnp.inf, jnp.float32)
    runidx = jnp.zeros((BM, 1), jnp.int32)
    for c in range(NCH):
        dot2 = lax.dot_general(
            x2, emb_ref[pl.ds(c * CHN, CHN), :],
            dimension_numbers=(((1,), (1,)), ((), ())),
            preferred_element_type=jnp.float32)
        dist = (xsq - dot2) + esq_ref[:, pl.ds(c * CHN, CHN)]
        cmin = jnp.min(dist, axis=1, keepdims=True)
        fidx = jnp.min(
            jnp.where(dist == cmin, locf_ref[:, pl.ds(c * CHN, CHN)],
                      jnp.inf),
            axis=1, keepdims=True)
        cidx = fidx.astype(jnp.int32)
        take = cmin < runmin
        runidx = jnp.where(take, cidx, runidx)
        runmin = jnp.where(take, cmin, runmin)
    idx_ref[...] = runidx
    loss_ref[...] = loss_ref[...] + jnp.sum(runmin, axis=0, keepdims=True)

    @pl.when(i == NSTEPS - 1)
    def _fin():
        loss_ref[...] = loss_ref[...] * (2.0 / NUMEL)


_dist_argmin = pl.pallas_call(
    _dist_argmin_body,
    grid=(NSTEPS,),
    in_specs=[
        pl.BlockSpec((BM, DIM), lambda i: (i, 0)),
        pl.BlockSpec((CB, DIM), lambda i: (0, 0)),
        pl.BlockSpec((BM, 1), lambda i: (i, 0)),
        pl.BlockSpec((1, CB), lambda i: (0, 0)),
        pl.BlockSpec((1, CB), lambda i: (0, 0)),
    ],
    out_specs=[
        pl.BlockSpec((BM, 1), lambda i: (i, 0)),
        pl.BlockSpec((1, 1), lambda i: (0, 0)),
    ],
    out_shape=[
        jax.ShapeDtypeStruct((FLAT, 1), jnp.int32),
        jax.ShapeDtypeStruct((1, 1), jnp.float32),
    ],
)


_NW = 32
_BPW = FLAT // _NW
_CH = 128

_sc_mesh = plsc.VectorSubcoreMesh(core_axis_name="c", subcore_axis_name="s")


@functools.partial(
    pl.kernel,
    mesh=_sc_mesh,
    out_type=jax.ShapeDtypeStruct((FLAT, DIM), jnp.float32),
    scratch_types=[
        pltpu.VMEM((_CH,), jnp.int32),
        pltpu.VMEM((_CH,), jnp.int32),
        pltpu.VMEM((_CH, DIM), jnp.float32),
        pltpu.VMEM((_CH, DIM), jnp.float32),
        pltpu.SemaphoreType.DMA,
        pltpu.SemaphoreType.DMA,
        pltpu.SemaphoreType.DMA,
        pltpu.SemaphoreType.DMA,
    ],
)
def _sc_gather(emb_hbm, idx_hbm, out_hbm, idxa, idxb, rowsa, rowsb,
               sa, sb, soa, sob):
    wid = lax.axis_index("s") * 2 + lax.axis_index("c")
    base = wid * _BPW
    c0, c1, c2, c3 = base, base + _CH, base + 2 * _CH, base + 3 * _CH
    pltpu.sync_copy(idx_hbm.at[pl.ds(c0, _CH)], idxa)
    g0 = pltpu.async_copy(emb_hbm.at[idxa], rowsa, sa)
    pltpu.sync_copy(idx_hbm.at[pl.ds(c1, _CH)], idxb)
    g1 = pltpu.async_copy(emb_hbm.at[idxb], rowsb, sb)
    g0.wait()
    o0 = pltpu.async_copy(rowsa, out_hbm.at[pl.ds(c0, _CH)], soa)
    g1.wait()
    o1 = pltpu.async_copy(rowsb, out_hbm.at[pl.ds(c1, _CH)], sob)
    o0.wait()
    pltpu.sync_copy(idx_hbm.at[pl.ds(c2, _CH)], idxa)
    g2 = pltpu.async_copy(emb_hbm.at[idxa], rowsa, sa)
    o1.wait()
    pltpu.sync_copy(idx_hbm.at[pl.ds(c3, _CH)], idxb)
    g3 = pltpu.async_copy(emb_hbm.at[idxb], rowsb, sb)
    g2.wait()
    o2 = pltpu.async_copy(rowsa, out_hbm.at[pl.ds(c2, _CH)], soa)
    g3.wait()
    o3 = pltpu.async_copy(rowsb, out_hbm.at[pl.ds(c3, _CH)], sob)
    o2.wait()
    o3.wait()


def kernel(x, embedding):
    flat = x.reshape(FLAT, DIM)
    xsq = jnp.sum(flat * flat, axis=1, keepdims=True)
    esq = jnp.sum(embedding * embedding, axis=1).reshape(1, CB)
    locf = jnp.arange(CB, dtype=jnp.float32).reshape(1, CB)
    idx_col, loss2d = _dist_argmin(flat, embedding, xsq, esq, locf)
    idx_flat = idx_col.reshape(FLAT)
    quant = _sc_gather(embedding, idx_flat)
    quantized_st = quant.reshape(x.shape)
    indices = idx_flat.reshape(x.shape[:-1])
    loss = loss2d.reshape(())
    return quantized_st, indices, loss
